# baseline (speedup 1.0000x reference)
import functools

import jax
import jax.numpy as jnp
from jax import lax
from jax.experimental import pallas as pl
from jax.experimental.pallas import tpu as pltpu
from jax.experimental.pallas import tpu_sc as plsc

B = 4096
VOCAB = 100000
EMBED = 64
NCAT = 8
DOUT = (NCAT + 1) * EMBED

_info = plsc.get_sparse_core_info()
_NC, _NS = _info.num_cores, _info.num_subcores
_NW = _NC * _NS
_BPW = B // _NW
_G = _BPW // 4


def _make_kernel():
    mesh = plsc.VectorSubcoreMesh(core_axis_name="c", subcore_axis_name="s")

    @functools.partial(
        pl.kernel,
        mesh=mesh,
        out_type=jax.ShapeDtypeStruct((DOUT, B), jnp.float32),
        scratch_types=[
            pltpu.VMEM((NCAT * _BPW,), jnp.int32),
            pltpu.VMEM((NCAT * _BPW,), jnp.int32),
            pltpu.VMEM((NCAT * _BPW,), jnp.int32),
            pltpu.VMEM((_G, 4, 2 * EMBED), jnp.float32),
            pltpu.VMEM((DOUT, _BPW), jnp.float32),
            pltpu.SemaphoreType.DMA,
        ],
        compiler_params=pltpu.CompilerParams(needs_layout_passes=False),
    )
    def body(cat_0, cat_1, cat_2, cat_3, cat_4, cat_5, cat_6, cat_7,
             dense_t, table_0, table_1, table_2, table_3, table_4, table_5,
             table_6, table_7, out_t, gid_v, sub_v, lb_v, blk_v, stage_v,
             sem):
        cats = [cat_0, cat_1, cat_2, cat_3, cat_4, cat_5, cat_6, cat_7]
        tables = [table_0, table_1, table_2, table_3, table_4, table_5,
                  table_6, table_7]
        wid = lax.axis_index("s") * _NC + lax.axis_index("c")
        base = wid * _BPW

        for i in range(NCAT):
            pltpu.sync_copy(cats[i].at[pl.ds(base, _BPW)],
                            gid_v.at[pl.ds(i * _BPW, _BPW)])
        for k in range(NCAT * _BPW // 16):
            v = gid_v[pl.ds(k * 16, 16)]
            sub_v[pl.ds(k * 16, 16)] = lax.shift_right_logical(
                lax.bitwise_and(v, 7), 1)
            lb_v[pl.ds(k * 16, 16)] = lax.bitwise_and(v, 1) * 64
            gid_v[pl.ds(k * 16, 16)] = lax.shift_right_logical(v, 3)

        pltpu.sync_copy(dense_t.at[:, pl.ds(base, _BPW)],
                        stage_v.at[pl.ds(NCAT * EMBED, EMBED), :])

        iota = lax.iota(jnp.int32, 16)

        for i in range(NCAT):
            def pass_body(p, _, i=i, tbl=tables[i]):
                off = i * _BPW + p * _G
                pltpu.sync_copy(tbl.at[gid_v.at[pl.ds(off, _G)]], blk_v)
                subs = [sub_v[pl.ds(off + c * 16, 16)] for c in range(_G // 16)]
                lbs = [lb_v[pl.ds(off + c * 16, 16)] for c in range(_G // 16)]
                for e in range(EMBED):
                    row = i * EMBED + e
                    for c in range(_G // 16):
                        vals = plsc.load_gather(
                            blk_v, [iota + c * 16, subs[c], lbs[c] + e])
                        stage_v[row, pl.ds(p * _G + c * 16, 16)] = vals
                return 0

            lax.fori_loop(0, _BPW // _G, pass_body, 0)

        pltpu.sync_copy(stage_v, out_t.at[:, pl.ds(base, _BPW)])

    return body


_kernel_call = _make_kernel()


def kernel(cat_0, cat_1, cat_2, cat_3, cat_4, cat_5, cat_6, cat_7, dense,
           table_0, table_1, table_2, table_3, table_4, table_5, table_6,
           table_7):
    tabs = [t.reshape(VOCAB // 8, 4, 2 * EMBED)
            for t in (table_0, table_1, table_2, table_3, table_4, table_5,
                      table_6, table_7)]
    out_t = _kernel_call(cat_0, cat_1, cat_2, cat_3, cat_4, cat_5, cat_6,
                         cat_7, dense.T, *tabs)
    return out_t.T


# speedup vs baseline: 1.1790x; 1.1790x over previous
import functools

import jax
import jax.numpy as jnp
from jax import lax
from jax.experimental import pallas as pl
from jax.experimental.pallas import tpu as pltpu
from jax.experimental.pallas import tpu_sc as plsc

B = 4096
VOCAB = 100000
EMBED = 64
NCAT = 8
DOUT = (NCAT + 1) * EMBED

_info = plsc.get_sparse_core_info()
_NC, _NS = _info.num_cores, _info.num_subcores
_NW = _NC * _NS
_BPW = B // _NW


def _make_kernel():
    mesh = plsc.VectorSubcoreMesh(core_axis_name="c", subcore_axis_name="s")

    @functools.partial(
        pl.kernel,
        mesh=mesh,
        out_type=jax.ShapeDtypeStruct((B, DOUT), jnp.float32),
        scratch_types=(
            [pltpu.VMEM((_BPW,), jnp.int32) for _ in range(NCAT)]
            + [pltpu.VMEM((_BPW, EMBED), jnp.float32) for _ in range(NCAT + 1)]
            + [pltpu.SemaphoreType.DMA, pltpu.SemaphoreType.DMA]
        ),
        compiler_params=pltpu.CompilerParams(use_tc_tiling_on_sc=False),
    )
    def body(cat_0, cat_1, cat_2, cat_3, cat_4, cat_5, cat_6, cat_7,
             dense, table_0, table_1, table_2, table_3, table_4, table_5,
             table_6, table_7, out,
             idx_0, idx_1, idx_2, idx_3, idx_4, idx_5, idx_6, idx_7,
             rows_0, rows_1, rows_2, rows_3, rows_4, rows_5, rows_6, rows_7,
             dense_v, sem_g, sem_w):
        cats = [cat_0, cat_1, cat_2, cat_3, cat_4, cat_5, cat_6, cat_7]
        tables = [table_0, table_1, table_2, table_3, table_4, table_5,
                  table_6, table_7]
        idxs = [idx_0, idx_1, idx_2, idx_3, idx_4, idx_5, idx_6, idx_7]
        rows = [rows_0, rows_1, rows_2, rows_3, rows_4, rows_5, rows_6,
                rows_7]

        wid = lax.axis_index("s") * _NC + lax.axis_index("c")
        base = wid * _BPW

        for i in range(NCAT):
            pltpu.sync_copy(cats[i].at[pl.ds(base, _BPW)], idxs[i])
        for i in range(NCAT):
            pltpu.make_async_copy(tables[i].at[idxs[i]], rows[i],
                                  sem_g).start()
        pltpu.make_async_copy(dense.at[pl.ds(base, _BPW)], dense_v,
                              sem_w).start()

        for i in range(NCAT):
            pltpu.make_async_copy(tables[i].at[idxs[i]], rows[i],
                                  sem_g).wait()
            pltpu.sync_copy(rows[i],
                            out.at[pl.ds(base, _BPW),
                                   pl.ds(i * EMBED, EMBED)])
        pltpu.make_async_copy(dense.at[pl.ds(base, _BPW)], dense_v,
                              sem_w).wait()
        pltpu.sync_copy(dense_v,
                        out.at[pl.ds(base, _BPW),
                               pl.ds(NCAT * EMBED, EMBED)])

    return body


_kernel_call = _make_kernel()


def kernel(cat_0, cat_1, cat_2, cat_3, cat_4, cat_5, cat_6, cat_7, dense,
           table_0, table_1, table_2, table_3, table_4, table_5, table_6,
           table_7):
    return _kernel_call(cat_0, cat_1, cat_2, cat_3, cat_4, cat_5, cat_6,
                        cat_7, dense, table_0, table_1, table_2, table_3,
                        table_4, table_5, table_6, table_7)


# speedup vs baseline: 1.3617x; 1.1550x over previous
import functools

import jax
import jax.numpy as jnp
from jax import lax
from jax.experimental import pallas as pl
from jax.experimental.pallas import tpu as pltpu
from jax.experimental.pallas import tpu_sc as plsc

B = 4096
VOCAB = 100000
EMBED = 64
NCAT = 8
DOUT = (NCAT + 1) * EMBED

_info = plsc.get_sparse_core_info()
_NC, _NS = _info.num_cores, _info.num_subcores
_NW = _NC * _NS
_BPW = B // _NW
_H = 8


def _make_kernel():
    mesh = plsc.VectorSubcoreMesh(core_axis_name="c", subcore_axis_name="s")

    @functools.partial(
        pl.kernel,
        mesh=mesh,
        out_type=jax.ShapeDtypeStruct((B, DOUT), jnp.float32),
        scratch_types=[
            pltpu.VMEM((NCAT * _BPW,), jnp.int32),
            pltpu.VMEM((2 * _H, 8, EMBED), jnp.float32),
            pltpu.VMEM((_BPW, DOUT), jnp.float32),
            pltpu.VMEM((_BPW, EMBED), jnp.float32),
            pltpu.SemaphoreType.DMA,
            pltpu.SemaphoreType.DMA,
            pltpu.SemaphoreType.DMA,
        ],
        compiler_params=pltpu.CompilerParams(needs_layout_passes=False),
    )
    def body(cat_0, cat_1, cat_2, cat_3, cat_4, cat_5, cat_6, cat_7,
             dense, table_0, table_1, table_2, table_3, table_4, table_5,
             table_6, table_7, out,
             idx_v, blk, stage, dense_v, sem_a, sem_b, sem_d):
        cats = [cat_0, cat_1, cat_2, cat_3, cat_4, cat_5, cat_6, cat_7]
        tables = [table_0, table_1, table_2, table_3, table_4, table_5,
                  table_6, table_7]

        wid = lax.axis_index("s") * _NC + lax.axis_index("c")
        base = wid * _BPW

        for i in range(NCAT):
            pltpu.sync_copy(cats[i].at[pl.ds(base, _BPW)],
                            idx_v.at[pl.ds(i * _BPW, _BPW)])
        pltpu.make_async_copy(dense.at[pl.ds(base, _BPW)], dense_v,
                              sem_d).start()

        def group_copy(tbl, v, slot, sem):
            g8 = pl.multiple_of((v >> 3) << 3, 8)
            return pltpu.make_async_copy(tbl.at[pl.ds(g8, 8), :],
                                         blk.at[slot], sem)

        def extract(v, k, t, slot):
            s = lax.bitwise_and(v, 7)
            for c in range(EMBED // 16):
                stage[k, pl.ds(t * EMBED + c * 16, 16)] = (
                    blk[slot, s, pl.ds(c * 16, 16)])

        n_pairs = _BPW // (2 * _H)

        for t in range(NCAT):
            tbl = tables[t]
            vec0 = idx_v[pl.ds(t * _BPW, 16)]
            for j in range(_H):
                group_copy(tbl, vec0[j], j, sem_a).start()
            for j in range(_H):
                group_copy(tbl, vec0[_H + j], _H + j, sem_b).start()

            def pair(p, _, tbl=tbl, t=t):
                row = 2 * p * _H
                vec = idx_v[pl.ds(t * _BPW + row, 16)]

                for j in range(_H):
                    group_copy(tbl, vec[j], j, sem_a).wait()
                for j in range(_H):
                    extract(vec[j], row + j, t, j)

                @pl.when(p < n_pairs - 1)
                def _():
                    nvec = idx_v[pl.ds(t * _BPW + row + 2 * _H, 16)]
                    for j in range(_H):
                        group_copy(tbl, nvec[j], j, sem_a).start()

                for j in range(_H):
                    group_copy(tbl, vec[_H + j], _H + j, sem_b).wait()
                for j in range(_H):
                    extract(vec[_H + j], row + _H + j, t, _H + j)

                @pl.when(p < n_pairs - 1)
                def _():
                    nvec = idx_v[pl.ds(t * _BPW + row + 2 * _H, 16)]
                    for j in range(_H):
                        group_copy(tbl, nvec[_H + j], _H + j, sem_b).start()

                return 0

            lax.fori_loop(0, n_pairs, pair, 0)

        pltpu.make_async_copy(dense.at[pl.ds(base, _BPW)], dense_v,
                              sem_d).wait()

        def dcopy(r, _):
            for c in range(EMBED // 16):
                stage[r, pl.ds(NCAT * EMBED + c * 16, 16)] = (
                    dense_v[r, pl.ds(c * 16, 16)])
            return 0

        lax.fori_loop(0, _BPW, dcopy, 0)

        pltpu.sync_copy(stage, out.at[pl.ds(base, _BPW)])

    return body


_kernel_call = _make_kernel()


def kernel(cat_0, cat_1, cat_2, cat_3, cat_4, cat_5, cat_6, cat_7, dense,
           table_0, table_1, table_2, table_3, table_4, table_5, table_6,
           table_7):
    return _kernel_call(cat_0, cat_1, cat_2, cat_3, cat_4, cat_5, cat_6,
                        cat_7, dense, table_0, table_1, table_2, table_3,
                        table_4, table_5, table_6, table_7)


# speedup vs baseline: 1.3959x; 1.0251x over previous
import functools

import jax
import jax.numpy as jnp
from jax import lax
from jax.experimental import pallas as pl
from jax.experimental.pallas import tpu as pltpu
from jax.experimental.pallas import tpu_sc as plsc

B = 4096
VOCAB = 100000
EMBED = 64
NCAT = 8
DOUT = (NCAT + 1) * EMBED

_info = plsc.get_sparse_core_info()
_NC, _NS = _info.num_cores, _info.num_subcores
_NW = _NC * _NS
_BPW = B // _NW
_H = 16


def _make_kernel():
    mesh = plsc.VectorSubcoreMesh(core_axis_name="c", subcore_axis_name="s")

    @functools.partial(
        pl.kernel,
        mesh=mesh,
        out_type=jax.ShapeDtypeStruct((B, DOUT), jnp.float32),
        scratch_types=[
            pltpu.VMEM((NCAT * _BPW,), jnp.int32),
            pltpu.VMEM((2 * _H, 8, EMBED), jnp.float32),
            pltpu.VMEM((_BPW, DOUT), jnp.float32),
            pltpu.VMEM((_BPW // 2, EMBED), jnp.float32),
            pltpu.SemaphoreType.DMA,
            pltpu.SemaphoreType.DMA,
            pltpu.SemaphoreType.DMA,
        ],
        compiler_params=pltpu.CompilerParams(needs_layout_passes=False),
    )
    def body(cat_0, cat_1, cat_2, cat_3, cat_4, cat_5, cat_6, cat_7,
             dense, table_0, table_1, table_2, table_3, table_4, table_5,
             table_6, table_7, out,
             idx_v, blk, stage, dense_v, sem_a, sem_b, sem_d):
        cats = [cat_0, cat_1, cat_2, cat_3, cat_4, cat_5, cat_6, cat_7]
        tables = [table_0, table_1, table_2, table_3, table_4, table_5,
                  table_6, table_7]

        wid = lax.axis_index("s") * _NC + lax.axis_index("c")
        base = wid * _BPW

        for i in range(NCAT):
            pltpu.sync_copy(cats[i].at[pl.ds(base, _BPW)],
                            idx_v.at[pl.ds(i * _BPW, _BPW)])
        pltpu.make_async_copy(dense.at[pl.ds(base, _BPW // 2)], dense_v,
                              sem_d).start()

        def group_copy(tbl, v, slot, sem):
            g8 = pl.multiple_of((v >> 3) << 3, 8)
            return pltpu.make_async_copy(tbl.at[pl.ds(g8, 8), :],
                                         blk.at[slot], sem)

        def extract(v, k, t, slot):
            s = lax.bitwise_and(v, 7)
            for c in range(EMBED // 16):
                stage[k, pl.ds(t * EMBED + c * 16, 16)] = (
                    blk[slot, s, pl.ds(c * 16, 16)])

        n_pairs = _BPW // (2 * _H)

        for t in range(NCAT):
            tbl = tables[t]
            vec_a0 = idx_v[pl.ds(t * _BPW, 16)]
            vec_b0 = idx_v[pl.ds(t * _BPW + _H, 16)]
            for j in range(_H):
                group_copy(tbl, vec_a0[j], j, sem_a).start()
            for j in range(_H):
                group_copy(tbl, vec_b0[j], _H + j, sem_b).start()

            def pair(p, _, tbl=tbl, t=t):
                row = 2 * p * _H
                vec_a = idx_v[pl.ds(t * _BPW + row, 16)]
                vec_b = idx_v[pl.ds(t * _BPW + row + _H, 16)]

                for j in range(_H):
                    group_copy(tbl, vec_a[j], j, sem_a).wait()
                for j in range(_H):
                    extract(vec_a[j], row + j, t, j)

                @pl.when(p < n_pairs - 1)
                def _():
                    nvec = idx_v[pl.ds(t * _BPW + row + 2 * _H, 16)]
                    for j in range(_H):
                        group_copy(tbl, nvec[j], j, sem_a).start()

                for j in range(_H):
                    group_copy(tbl, vec_b[j], _H + j, sem_b).wait()
                for j in range(_H):
                    extract(vec_b[j], row + _H + j, t, _H + j)

                @pl.when(p < n_pairs - 1)
                def _():
                    nvec = idx_v[pl.ds(t * _BPW + row + 3 * _H, 16)]
                    for j in range(_H):
                        group_copy(tbl, nvec[j], _H + j, sem_b).start()

                return 0

            lax.fori_loop(0, n_pairs, pair, 0)

        for h in range(2):
            pltpu.make_async_copy(
                dense.at[pl.ds(base + h * (_BPW // 2), _BPW // 2)],
                dense_v, sem_d).wait()
            h_off = h * (_BPW // 2)

            def dcopy(r, _, h_off=h_off):
                for c in range(EMBED // 16):
                    stage[h_off + r, pl.ds(NCAT * EMBED + c * 16, 16)] = (
                        dense_v[r, pl.ds(c * 16, 16)])
                return 0

            lax.fori_loop(0, _BPW // 2, dcopy, 0)
            if h == 0:
                pltpu.make_async_copy(
                    dense.at[pl.ds(base + _BPW // 2, _BPW // 2)],
                    dense_v, sem_d).start()

        pltpu.sync_copy(stage, out.at[pl.ds(base, _BPW)])

    return body


_kernel_call = _make_kernel()


def kernel(cat_0, cat_1, cat_2, cat_3, cat_4, cat_5, cat_6, cat_7, dense,
           table_0, table_1, table_2, table_3, table_4, table_5, table_6,
           table_7):
    return _kernel_call(cat_0, cat_1, cat_2, cat_3, cat_4, cat_5, cat_6,
                        cat_7, dense, table_0, table_1, table_2, table_3,
                        table_4, table_5, table_6, table_7)
